# baseline (speedup 1.0000x reference)
import functools

import jax
import jax.numpy as jnp
from jax import lax
from jax.experimental import pallas as pl
from jax.experimental.pallas import tpu as pltpu
from jax.experimental.pallas import tpu_sc as plsc

EMBED = 64
NW = 32
BCHUNK = 256
IDX_ROW = 128
STREAMS = BCHUNK // IDX_ROW


@jax.jit
def _sc_embedding_gather(tw, xT):
    F, BATCH = xT.shape
    b_per_w = BATCH // NW
    n_chunks = b_per_w // BCHUNK
    n_blocks = F * n_chunks
    assert n_blocks % 2 == 0

    mesh = plsc.VectorSubcoreMesh(core_axis_name="c", subcore_axis_name="s")

    @functools.partial(
        pl.kernel,
        mesh=mesh,
        compiler_params=pltpu.CompilerParams(
            use_tc_tiling_on_sc=False, needs_layout_passes=False
        ),
        out_type=jax.ShapeDtypeStruct((F, EMBED, BATCH), jnp.float32),
        scratch_types=[
            pltpu.VMEM((F, b_per_w), jnp.int32),
            pltpu.VMEM((BCHUNK, EMBED), jnp.float32),
            pltpu.VMEM((BCHUNK, EMBED), jnp.float32),
            pltpu.VMEM((EMBED, BCHUNK), jnp.float32),
            pltpu.VMEM((EMBED, BCHUNK), jnp.float32),
            pltpu.SemaphoreType.DMA,
            pltpu.SemaphoreType.DMA,
        ],
    )
    def k(tw_hbm, xT_hbm, z_hbm, idx_v, w0_v, w1_v, t0_v, t1_v, gsem, osem):
        cid = lax.axis_index("c")
        sid = lax.axis_index("s")
        wid = sid * 2 + cid
        bbase = wid * b_per_w
        pltpu.sync_copy(xT_hbm.at[:, pl.ds(bbase, b_per_w)], idx_v)

        lane = lax.iota(jnp.int32, 16)

        def fire_gathers(blk, w_v):
            f = lax.rem(blk, F)
            boff = lax.div(blk, F) * BCHUNK
            for j in range(STREAMS):
                pltpu.async_copy(
                    tw_hbm.at[idx_v.at[f].at[pl.ds(boff + j * IDX_ROW, IDX_ROW)]],
                    w_v.at[pl.ds(j * IDX_ROW, IDX_ROW)],
                    gsem,
                )

        def wait_gathers():
            for j in range(STREAMS):
                pltpu.make_async_copy(
                    tw_hbm.at[idx_v.at[0].at[pl.ds(0, IDX_ROW)]],
                    w0_v.at[pl.ds(0, IDX_ROW)],
                    gsem,
                ).wait()

        def wait_one_out():
            pltpu.make_async_copy(
                t0_v,
                z_hbm.at[0].at[:, pl.ds(bbase, BCHUNK)],
                osem,
            ).wait()

        diag = [lax.rem(lane + r, 16) for r in range(16)]

        def transpose_and_send(blk, w_v, t_v):
            f = lax.rem(blk, F)
            boff = lax.div(blk, F) * BCHUNK

            @pl.loop(0, BCHUNK, step=16)
            def _(l0):
                row_v = lane + l0
                for c0 in range(0, EMBED, 16):
                    cols = [diag[r] + c0 for r in range(16)]
                    vals = [
                        plsc.load_gather(w_v, [row_v, cols[r]])
                        for r in range(16)
                    ]
                    for r in range(16):
                        plsc.store_scatter(t_v, [cols[r], row_v], vals[r])

            pltpu.async_copy(
                t_v,
                z_hbm.at[f].at[:, pl.ds(bbase + boff, BCHUNK)],
                osem,
            )

        fire_gathers(0, w0_v)

        @pl.loop(0, n_blocks, step=2)
        def _(blk0):
            for par, (w_v, t_v) in enumerate(((w0_v, t0_v), (w1_v, t1_v))):
                blk = blk0 + par
                other_w = w1_v if par == 0 else w0_v

                @pl.when(blk + 1 < n_blocks)
                def _():
                    fire_gathers(blk + 1, other_w)

                wait_gathers()

                @pl.when(blk >= 2)
                def _():
                    wait_one_out()

                transpose_and_send(blk, w_v, t_v)

        wait_one_out()
        wait_one_out()

    return k(tw, xT)


def kernel(x, table):
    b, f = x.shape
    xT = x.T
    tw = jnp.pad(table, ((0, 0), (0, 2 * EMBED - table.shape[1])))
    z = _sc_embedding_gather(tw.reshape(2 * table.shape[0], EMBED), xT * 2)
    return z.transpose(2, 0, 1)


# speedup vs baseline: 1.1903x; 1.1903x over previous
import functools

import jax
import jax.numpy as jnp
from jax import lax
from jax.experimental import pallas as pl
from jax.experimental.pallas import tpu as pltpu
from jax.experimental.pallas import tpu_sc as plsc

EMBED = 64
NW = 32
BCHUNK = 256
IDX_ROW = 128
STREAMS = BCHUNK // IDX_ROW


@jax.jit
def _sc_embedding_gather(tw, xT):
    F, BATCH = xT.shape
    b_per_w = BATCH // NW
    n_chunks = b_per_w // BCHUNK
    n_blocks = F * n_chunks
    assert n_blocks % 2 == 0

    mesh = plsc.VectorSubcoreMesh(core_axis_name="c", subcore_axis_name="s")

    @functools.partial(
        pl.kernel,
        mesh=mesh,
        compiler_params=pltpu.CompilerParams(
            use_tc_tiling_on_sc=True, needs_layout_passes=False
        ),
        out_type=jax.ShapeDtypeStruct((F, EMBED, BATCH), jnp.float32),
        scratch_types=[
            pltpu.VMEM((F, b_per_w), jnp.int32),
            pltpu.VMEM((BCHUNK, 2 * EMBED), jnp.float32),
            pltpu.VMEM((BCHUNK, 2 * EMBED), jnp.float32),
            pltpu.VMEM((EMBED, BCHUNK), jnp.float32),
            pltpu.VMEM((EMBED, BCHUNK), jnp.float32),
            pltpu.SemaphoreType.DMA,
            pltpu.SemaphoreType.DMA,
        ],
    )
    def k(tw_hbm, xT_hbm, z_hbm, idx_v, w0_v, w1_v, t0_v, t1_v, gsem, osem):
        cid = lax.axis_index("c")
        sid = lax.axis_index("s")
        wid = sid * 2 + cid
        bbase = wid * b_per_w
        pltpu.sync_copy(xT_hbm.at[:, pl.ds(bbase, b_per_w)], idx_v)

        lane = lax.iota(jnp.int32, 16)

        def fire_gathers(blk, w_v):
            f = lax.rem(blk, F)
            boff = lax.div(blk, F) * BCHUNK
            for j in range(STREAMS):
                pltpu.async_copy(
                    tw_hbm.at[idx_v.at[f].at[pl.ds(boff + j * IDX_ROW, IDX_ROW)]],
                    w_v.at[pl.ds(j * IDX_ROW, IDX_ROW)],
                    gsem,
                )

        def wait_gathers():
            for j in range(STREAMS):
                pltpu.make_async_copy(
                    tw_hbm.at[idx_v.at[0].at[pl.ds(0, IDX_ROW)]],
                    w0_v.at[pl.ds(0, IDX_ROW)],
                    gsem,
                ).wait()

        def wait_one_out():
            pltpu.make_async_copy(
                t0_v,
                z_hbm.at[0].at[:, pl.ds(bbase, BCHUNK)],
                osem,
            ).wait()

        diag = [lax.rem(lane + r, 16) for r in range(16)]

        def transpose_and_send(blk, w_v, t_v):
            f = lax.rem(blk, F)
            boff = lax.div(blk, F) * BCHUNK

            @pl.loop(0, BCHUNK, step=16)
            def _(l0):
                row_v = lane + l0
                for c0 in range(0, EMBED, 16):
                    cols = [diag[r] + c0 for r in range(16)]
                    vals = [
                        plsc.load_gather(w_v, [row_v, cols[r]])
                        for r in range(16)
                    ]
                    for r in range(16):
                        plsc.store_scatter(t_v, [cols[r], row_v], vals[r])

            pltpu.async_copy(
                t_v,
                z_hbm.at[f].at[:, pl.ds(bbase + boff, BCHUNK)],
                osem,
            )

        fire_gathers(0, w0_v)

        @pl.loop(0, n_blocks, step=2)
        def _(blk0):
            for par, (w_v, t_v) in enumerate(((w0_v, t0_v), (w1_v, t1_v))):
                blk = blk0 + par
                other_w = w1_v if par == 0 else w0_v

                @pl.when(blk + 1 < n_blocks)
                def _():
                    fire_gathers(blk + 1, other_w)

                wait_gathers()

                @pl.when(blk >= 2)
                def _():
                    wait_one_out()

                transpose_and_send(blk, w_v, t_v)

        wait_one_out()
        wait_one_out()

    return k(tw, xT)


def kernel(x, table):
    b, f = x.shape
    xT = x.T
    tw = jnp.pad(table, ((0, 0), (0, 2 * EMBED - table.shape[1])))
    z = _sc_embedding_gather(tw, xT)
    return z.transpose(2, 0, 1)
